# baseline (speedup 1.0000x reference)
import functools

import jax
import jax.numpy as jnp
from jax import lax
from jax.experimental import pallas as pl
from jax.experimental.pallas import tpu as pltpu
from jax.experimental.pallas import tpu_sc as plsc

BATCH = 16384
EMBED = 16
NUM_CORES = 2
NUM_SUBCORES = 16
NW = NUM_CORES * NUM_SUBCORES
B_PER_W = BATCH // NW
CHUNK = 128
N_CHUNKS = B_PER_W // CHUNK

MAX_RATING = 5.0
MIN_RATING = 0.5


def _body(uid_hbm, mid_hbm, utab_hbm, mtab_hbm, w_hbm, out_hbm,
          uidx, midx, urows, mrows, wv, outv, usem, msem):
    wid = lax.axis_index("s") * NUM_CORES + lax.axis_index("c")
    base = wid * B_PER_W

    pltpu.sync_copy(uid_hbm.at[pl.ds(base, B_PER_W)], uidx)
    pltpu.sync_copy(mid_hbm.at[pl.ds(base, B_PER_W)], midx)
    pltpu.sync_copy(w_hbm, wv)

    copies = []
    for c in range(N_CHUNKS):
        sl = pl.ds(c * CHUNK, CHUNK)
        copies.append(pltpu.async_copy(utab_hbm.at[uidx.at[sl]], urows.at[sl], usem))
        copies.append(pltpu.async_copy(mtab_hbm.at[midx.at[sl]], mrows.at[sl], msem))

    wuv = wv[pl.ds(0, EMBED)]
    wmv = wv[pl.ds(EMBED, EMBED)]
    bv = wv[pl.ds(2 * EMBED, EMBED)]

    for cp in copies:
        cp.wait()

    zero = jnp.zeros((EMBED,), jnp.float32)
    lanes = lax.iota(jnp.int32, EMBED)

    def grp_body(g, carry):
        off = pl.multiple_of(g * EMBED, EMBED)
        acc = zero
        for k in range(EMBED):
            u = urows[off + k]
            m = mrows[off + k]
            s = jnp.sum(u * wuv + m * wmv)
            acc = jnp.where(lanes == k, s, acc)
        y = (MAX_RATING - MIN_RATING) / (1.0 + jnp.exp(-(acc + bv))) + MIN_RATING
        outv[pl.ds(off, EMBED)] = y
        return carry

    lax.fori_loop(0, B_PER_W // EMBED, grp_body, 0)

    pltpu.sync_copy(outv, out_hbm.at[pl.ds(base, B_PER_W)])


@functools.partial(
    pl.kernel,
    mesh=plsc.VectorSubcoreMesh(core_axis_name="c", subcore_axis_name="s"),
    out_type=jax.ShapeDtypeStruct((BATCH,), jnp.float32),
    compiler_params=pltpu.CompilerParams(
        needs_layout_passes=False, use_tc_tiling_on_sc=False
    ),
    scratch_types=[
        pltpu.VMEM((B_PER_W,), jnp.int32),
        pltpu.VMEM((B_PER_W,), jnp.int32),
        pltpu.VMEM((B_PER_W, EMBED), jnp.float32),
        pltpu.VMEM((B_PER_W, EMBED), jnp.float32),
        pltpu.VMEM((3 * EMBED,), jnp.float32),
        pltpu.VMEM((B_PER_W,), jnp.float32),
        pltpu.SemaphoreType.DMA,
        pltpu.SemaphoreType.DMA,
    ],
)
def _sc_model(*refs):
    _body(*refs)


def kernel(user_ids, movie_ids, user_table, movie_table, W, b):
    uid = user_ids.astype(jnp.int32)
    mid = movie_ids.astype(jnp.int32)
    wflat = W.reshape(-1).astype(jnp.float32)
    wall = jnp.concatenate([wflat, jnp.broadcast_to(b.astype(jnp.float32), (EMBED,))])
    return _sc_model(uid, mid, user_table, movie_table, wall)
